# Initial kernel scaffold; baseline (speedup 1.0000x reference)
import jax
import jax.numpy as jnp
from jax.experimental import pallas as pl


def kernel(y_pred, y_true, lut):
    raise NotImplementedError("write your pallas kernel here")


# speedup vs baseline: 632.3553x; 632.3553x over previous
import functools

import jax
import jax.numpy as jnp
from jax import lax
from jax.experimental import pallas as pl
from jax.experimental.pallas import tpu as pltpu
from jax.experimental.pallas import tpu_sc as plsc

SDF_MIN = -7.0
SDF_MAX = 7.0
N_BINS = 256

_NC = 2
_NS = 16
_NW = _NC * _NS
_LANES = 16
_CHUNK = 8192


def _wse_partials(y_pred_flat, y_true_flat, lut):
    n = y_pred_flat.shape[0]
    per_w = n // _NW
    n_chunks = per_w // _CHUNK
    mesh = plsc.VectorSubcoreMesh(core_axis_name="c", subcore_axis_name="s")

    @functools.partial(
        pl.kernel,
        mesh=mesh,
        out_type=jax.ShapeDtypeStruct((_NW, _LANES), jnp.float32),
        scratch_types=[
            pltpu.VMEM((N_BINS,), jnp.float32),
            pltpu.VMEM((_CHUNK,), jnp.float32),
            pltpu.VMEM((_CHUNK,), jnp.float32),
            pltpu.VMEM((_LANES,), jnp.float32),
        ],
        compiler_params=pltpu.CompilerParams(needs_layout_passes=False),
    )
    def k(pred_hbm, true_hbm, lut_hbm, out_hbm, lut_v, pred_v, true_v, acc_v):
        wid = lax.axis_index("s") * _NC + lax.axis_index("c")
        base = wid * per_w
        pltpu.sync_copy(lut_hbm, lut_v)

        scale = 1.0 / (SDF_MAX - SDF_MIN)

        def chunk_body(g, acc):
            off = base + g * _CHUNK
            pltpu.sync_copy(pred_hbm.at[pl.ds(off, _CHUNK)], pred_v)
            pltpu.sync_copy(true_hbm.at[pl.ds(off, _CHUNK)], true_v)

            def vec_body(v, acc):
                t = true_v[pl.ds(v * _LANES, _LANES)]
                p = pred_v[pl.ds(v * _LANES, _LANES)]
                c = jnp.minimum(jnp.maximum(t, SDF_MIN), SDF_MAX)
                u = (c - SDF_MIN) * scale
                f = u * float(N_BINS - 1)
                idx = (f + 0.5).astype(jnp.int32)
                w = plsc.load_gather(lut_v, [idx])
                d = p - t
                return acc + w * d * d

            return lax.fori_loop(0, _CHUNK // _LANES, vec_body, acc)

        acc = lax.fori_loop(0, n_chunks, chunk_body,
                            jnp.zeros((_LANES,), jnp.float32))
        acc_v[...] = acc
        pltpu.sync_copy(acc_v, out_hbm.at[wid])

    return k(y_pred_flat, y_true_flat, lut)


def kernel(y_pred, y_true, lut):
    n = y_pred.size
    partials = _wse_partials(y_pred.reshape(-1), y_true.reshape(-1), lut)
    return partials.sum() / n


# speedup vs baseline: 1102.7837x; 1.7439x over previous
import functools

import jax
import jax.numpy as jnp
from jax import lax
from jax.experimental import pallas as pl
from jax.experimental.pallas import tpu as pltpu
from jax.experimental.pallas import tpu_sc as plsc

SDF_MIN = -7.0
SDF_MAX = 7.0
N_BINS = 256

_NC = 2
_NS = 16
_NW = _NC * _NS
_LANES = 16
_CHUNK = 16384


def _wse_partials(y_pred_flat, y_true_flat, lut):
    n = y_pred_flat.shape[0]
    per_w = n // _NW
    assert per_w % _CHUNK == 0 and (per_w // _CHUNK) % 2 == 0
    n_chunks = per_w // _CHUNK
    mesh = plsc.VectorSubcoreMesh(core_axis_name="c", subcore_axis_name="s")

    @functools.partial(
        pl.kernel,
        mesh=mesh,
        out_type=jax.ShapeDtypeStruct((_NW, _LANES), jnp.float32),
        scratch_types=[
            pltpu.VMEM((N_BINS,), jnp.float32),
            pltpu.VMEM((_CHUNK,), jnp.float32),
            pltpu.VMEM((_CHUNK,), jnp.float32),
            pltpu.VMEM((_CHUNK,), jnp.float32),
            pltpu.VMEM((_CHUNK,), jnp.float32),
            pltpu.VMEM((_LANES,), jnp.float32),
            pltpu.SemaphoreType.DMA,
            pltpu.SemaphoreType.DMA,
            pltpu.SemaphoreType.DMA,
            pltpu.SemaphoreType.DMA,
        ],
        compiler_params=pltpu.CompilerParams(needs_layout_passes=False),
    )
    def k(pred_hbm, true_hbm, lut_hbm, out_hbm,
          lut_v, pred0, true0, pred1, true1, acc_v, sp0, st0, sp1, st1):
        wid = lax.axis_index("s") * _NC + lax.axis_index("c")
        base = wid * per_w
        pltpu.sync_copy(lut_hbm, lut_v)

        off_cap = n - _CHUNK
        scale = 1.0 / (SDF_MAX - SDF_MIN)

        def issue(pred_v, true_v, sp, st, off):
            pltpu.async_copy(pred_hbm.at[pl.ds(off, _CHUNK)], pred_v, sp)
            pltpu.async_copy(true_hbm.at[pl.ds(off, _CHUNK)], true_v, st)

        def wait(pred_v, true_v, sp, st):
            pltpu.make_async_copy(pred_hbm.at[pl.ds(0, _CHUNK)], pred_v, sp).wait()
            pltpu.make_async_copy(true_hbm.at[pl.ds(0, _CHUNK)], true_v, st).wait()

        def compute(pred_v, true_v, acc):
            def vec_body(v, acc):
                t = true_v[pl.ds(v * _LANES, _LANES)]
                p = pred_v[pl.ds(v * _LANES, _LANES)]
                c = jnp.minimum(jnp.maximum(t, SDF_MIN), SDF_MAX)
                u = (c - SDF_MIN) * scale
                f = u * float(N_BINS - 1)
                idx = (f + 0.5).astype(jnp.int32)
                w = plsc.load_gather(lut_v, [idx])
                d = p - t
                return acc + w * d * d
            return lax.fori_loop(0, _CHUNK // _LANES, vec_body, acc)

        issue(pred0, true0, sp0, st0, base)

        def body(i, acc):
            g0 = 2 * i
            issue(pred1, true1, sp1, st1, base + (g0 + 1) * _CHUNK)
            wait(pred0, true0, sp0, st0)
            acc = compute(pred0, true0, acc)
            off2 = jnp.minimum(base + (g0 + 2) * _CHUNK, off_cap)
            issue(pred0, true0, sp0, st0, off2)
            wait(pred1, true1, sp1, st1)
            acc = compute(pred1, true1, acc)
            return acc

        acc = lax.fori_loop(0, n_chunks // 2, body,
                            jnp.zeros((_LANES,), jnp.float32))
        wait(pred0, true0, sp0, st0)
        acc_v[...] = acc
        pltpu.sync_copy(acc_v, out_hbm.at[wid])

    return k(y_pred_flat, y_true_flat, lut)


def kernel(y_pred, y_true, lut):
    n = y_pred.size
    partials = _wse_partials(y_pred.reshape(-1), y_true.reshape(-1), lut)
    return partials.sum() / n


# speedup vs baseline: 1536.0758x; 1.3929x over previous
import functools

import jax
import jax.numpy as jnp
from jax import lax
from jax.experimental import pallas as pl
from jax.experimental.pallas import tpu as pltpu
from jax.experimental.pallas import tpu_sc as plsc

SDF_MIN = -7.0
SDF_MAX = 7.0
N_BINS = 256

_NC = 2
_NS = 16
_NW = _NC * _NS
_LANES = 16
_CHUNK = 16384


def _wse_partials(y_pred_flat, y_true_flat, lut):
    n = y_pred_flat.shape[0]
    per_w = n // _NW
    assert per_w % _CHUNK == 0 and (per_w // _CHUNK) % 2 == 0
    n_chunks = per_w // _CHUNK
    mesh = plsc.VectorSubcoreMesh(core_axis_name="c", subcore_axis_name="s")

    @functools.partial(
        pl.kernel,
        mesh=mesh,
        out_type=jax.ShapeDtypeStruct((_NW, _LANES), jnp.float32),
        scratch_types=[
            pltpu.VMEM((N_BINS,), jnp.float32),
            pltpu.VMEM((_CHUNK,), jnp.float32),
            pltpu.VMEM((_CHUNK,), jnp.float32),
            pltpu.VMEM((_CHUNK,), jnp.float32),
            pltpu.VMEM((_CHUNK,), jnp.float32),
            pltpu.VMEM((_LANES,), jnp.float32),
            pltpu.SemaphoreType.DMA,
            pltpu.SemaphoreType.DMA,
            pltpu.SemaphoreType.DMA,
            pltpu.SemaphoreType.DMA,
        ],
        compiler_params=pltpu.CompilerParams(needs_layout_passes=False),
    )
    def k(pred_hbm, true_hbm, lut_hbm, out_hbm,
          lut_v, pred0, true0, pred1, true1, acc_v, sp0, st0, sp1, st1):
        wid = lax.axis_index("s") * _NC + lax.axis_index("c")
        base = wid * per_w
        pltpu.sync_copy(lut_hbm, lut_v)

        off_cap = n - _CHUNK
        scale = 1.0 / (SDF_MAX - SDF_MIN)

        def issue(pred_v, true_v, sp, st, off):
            pltpu.async_copy(pred_hbm.at[pl.ds(off, _CHUNK)], pred_v, sp)
            pltpu.async_copy(true_hbm.at[pl.ds(off, _CHUNK)], true_v, st)

        def wait(pred_v, true_v, sp, st):
            pltpu.make_async_copy(pred_hbm.at[pl.ds(0, _CHUNK)], pred_v, sp).wait()
            pltpu.make_async_copy(true_hbm.at[pl.ds(0, _CHUNK)], true_v, st).wait()

        bin_a = float(N_BINS - 1) * scale
        bin_b = -SDF_MIN * float(N_BINS - 1) * scale + 0.5

        def compute(pred_v, true_v, accs):
            def one(v, acc):
                t = true_v[pl.ds(v, _LANES)]
                p = pred_v[pl.ds(v, _LANES)]
                c = jnp.minimum(jnp.maximum(t, SDF_MIN), SDF_MAX)
                idx = (c * bin_a + bin_b).astype(jnp.int32)
                w = plsc.load_gather(lut_v, [idx])
                d = p - t
                return acc + w * d * d

            @plsc.parallel_loop(0, _CHUNK, step=4 * _LANES, unroll=2,
                                carry=accs)
            def body(v, accs):
                a0, a1, a2, a3 = accs
                a0 = one(v, a0)
                a1 = one(v + _LANES, a1)
                a2 = one(v + 2 * _LANES, a2)
                a3 = one(v + 3 * _LANES, a3)
                return (a0, a1, a2, a3)

            return body

        issue(pred0, true0, sp0, st0, base)

        def body(i, accs):
            g0 = 2 * i
            issue(pred1, true1, sp1, st1, base + (g0 + 1) * _CHUNK)
            wait(pred0, true0, sp0, st0)
            accs = compute(pred0, true0, accs)
            off2 = jnp.minimum(base + (g0 + 2) * _CHUNK, off_cap)
            issue(pred0, true0, sp0, st0, off2)
            wait(pred1, true1, sp1, st1)
            accs = compute(pred1, true1, accs)
            return accs

        z = jnp.zeros((_LANES,), jnp.float32)
        accs = lax.fori_loop(0, n_chunks // 2, body, (z, z, z, z))
        wait(pred0, true0, sp0, st0)
        a0, a1, a2, a3 = accs
        acc_v[...] = (a0 + a1) + (a2 + a3)
        pltpu.sync_copy(acc_v, out_hbm.at[wid])

    return k(y_pred_flat, y_true_flat, lut)


def kernel(y_pred, y_true, lut):
    n = y_pred.size
    partials = _wse_partials(y_pred.reshape(-1), y_true.reshape(-1), lut)
    return partials.sum() / n


# speedup vs baseline: 1754.3663x; 1.1421x over previous
import functools

import jax
import jax.numpy as jnp
from jax import lax
from jax.experimental import pallas as pl
from jax.experimental.pallas import tpu as pltpu
from jax.experimental.pallas import tpu_sc as plsc

SDF_MIN = -7.0
SDF_MAX = 7.0
N_BINS = 256

_NC = 2
_NS = 16
_NW = _NC * _NS
_LANES = 16
_CHUNK = 16384


def _wse_partials(y_pred_flat, y_true_flat, lut):
    n = y_pred_flat.shape[0]
    per_w = n // _NW
    assert per_w % _CHUNK == 0 and (per_w // _CHUNK) % 2 == 0
    n_chunks = per_w // _CHUNK
    mesh = plsc.VectorSubcoreMesh(core_axis_name="c", subcore_axis_name="s")

    @functools.partial(
        pl.kernel,
        mesh=mesh,
        out_type=jax.ShapeDtypeStruct((_NW, _LANES), jnp.float32),
        scratch_types=[
            pltpu.VMEM((N_BINS,), jnp.float32),
            pltpu.VMEM((_CHUNK,), jnp.float32),
            pltpu.VMEM((_CHUNK,), jnp.float32),
            pltpu.VMEM((_CHUNK,), jnp.float32),
            pltpu.VMEM((_CHUNK,), jnp.float32),
            pltpu.VMEM((_LANES,), jnp.float32),
            pltpu.SemaphoreType.DMA,
            pltpu.SemaphoreType.DMA,
            pltpu.SemaphoreType.DMA,
            pltpu.SemaphoreType.DMA,
        ],
        compiler_params=pltpu.CompilerParams(needs_layout_passes=False),
    )
    def k(pred_hbm, true_hbm, lut_hbm, out_hbm,
          lut_v, pred0, true0, pred1, true1, acc_v, sp0, st0, sp1, st1):
        wid = lax.axis_index("s") * _NC + lax.axis_index("c")
        base = wid * per_w
        pltpu.sync_copy(lut_hbm, lut_v)

        off_cap = n - _CHUNK
        scale = 1.0 / (SDF_MAX - SDF_MIN)

        def issue(pred_v, true_v, sp, st, off):
            pltpu.async_copy(pred_hbm.at[pl.ds(off, _CHUNK)], pred_v, sp)
            pltpu.async_copy(true_hbm.at[pl.ds(off, _CHUNK)], true_v, st)

        def wait(pred_v, true_v, sp, st):
            pltpu.make_async_copy(pred_hbm.at[pl.ds(0, _CHUNK)], pred_v, sp).wait()
            pltpu.make_async_copy(true_hbm.at[pl.ds(0, _CHUNK)], true_v, st).wait()

        bin_a = float(N_BINS - 1) * scale
        bin_b = -SDF_MIN * float(N_BINS - 1) * scale + 0.5

        def compute(pred_v, true_v, accs):
            def one(v, acc):
                t = true_v[pl.ds(v, _LANES)]
                p = pred_v[pl.ds(v, _LANES)]
                c = jnp.minimum(jnp.maximum(t, SDF_MIN), SDF_MAX)
                idx = (c * bin_a + bin_b).astype(jnp.int32)
                w = plsc.load_gather(lut_v, [idx])
                d = p - t
                return acc + w * d * d

            @plsc.parallel_loop(0, _CHUNK, step=4 * _LANES, unroll=4,
                                carry=accs)
            def body(v, accs):
                a0, a1, a2, a3 = accs
                a0 = one(v, a0)
                a1 = one(v + _LANES, a1)
                a2 = one(v + 2 * _LANES, a2)
                a3 = one(v + 3 * _LANES, a3)
                return (a0, a1, a2, a3)

            return body

        issue(pred0, true0, sp0, st0, base)

        def body(i, accs):
            g0 = 2 * i
            issue(pred1, true1, sp1, st1, base + (g0 + 1) * _CHUNK)
            wait(pred0, true0, sp0, st0)
            accs = compute(pred0, true0, accs)
            off2 = jnp.minimum(base + (g0 + 2) * _CHUNK, off_cap)
            issue(pred0, true0, sp0, st0, off2)
            wait(pred1, true1, sp1, st1)
            accs = compute(pred1, true1, accs)
            return accs

        z = jnp.zeros((_LANES,), jnp.float32)
        accs = lax.fori_loop(0, n_chunks // 2, body, (z, z, z, z))
        wait(pred0, true0, sp0, st0)
        a0, a1, a2, a3 = accs
        acc_v[...] = (a0 + a1) + (a2 + a3)
        pltpu.sync_copy(acc_v, out_hbm.at[wid])

    return k(y_pred_flat, y_true_flat, lut)


def kernel(y_pred, y_true, lut):
    n = y_pred.size
    partials = _wse_partials(y_pred.reshape(-1), y_true.reshape(-1), lut)
    return partials.sum() / n
